# baseline (speedup 1.0000x reference)
import functools

import jax
import jax.numpy as jnp
from jax import lax
from jax.experimental import pallas as pl
from jax.experimental.pallas import tpu as pltpu
from jax.experimental.pallas import tpu_sc as plsc

_N = 10000
_NNZ = 320000
_D = 128
_DH = _D // 2
_AUX = 16
_NC = 2
_NS = 16
_K = 128
_PS = _NNZ // _NS
_CH = 160
_CHA = _CH + 4
_PSP = _CHA * _K
_NB = 4
_RPS = 632
_NP = _RPS * _NS


def _sc_segment_pass(gather_aux: bool):
  mesh = plsc.VectorSubcoreMesh(core_axis_name="c", subcore_axis_name="s")
  out_type = (
      jax.ShapeDtypeStruct((_NC, _NP, _DH), jnp.float32),
      jax.ShapeDtypeStruct((_NC, _NP, _AUX), jnp.float32),
  )
  scratch = [
      pltpu.VMEM((_CHA, _K), jnp.int32),
      pltpu.VMEM((_CHA, _K), jnp.int32),
      [pltpu.VMEM((_K, _DH), jnp.float32)] * _NB,
      [pltpu.VMEM((_K, _AUX), jnp.float32)] * 2,
      pltpu.VMEM_SHARED((_NP, _DH), jnp.float32),
      pltpu.VMEM_SHARED((_NP, _AUX), jnp.float32),
      [pltpu.SemaphoreType.DMA] * _NB,
      [pltpu.SemaphoreType.DMA] * _NB,
      [pltpu.SemaphoreType.DMA] * 2,
      [pltpu.SemaphoreType.DMA] * 2,
  ]

  @functools.partial(
      pl.kernel, out_type=out_type, mesh=mesh, scratch_types=scratch,
      compiler_params=pltpu.CompilerParams(use_tc_tiling_on_sc=False))
  def run(tbl0, tbl1, auxtbl, gidx, sidx, zmain, zaux, onesbuf,
          out_main, out_aux,
          gv, sv, bufs, xbufs, accm, acca, sem_g, sem_s, sem_xg, sem_xs):
    c = lax.axis_index("c")
    s = lax.axis_index("s")
    pltpu.sync_copy(zmain, accm.at[pl.ds(s * _RPS, _RPS)])
    pltpu.sync_copy(zaux, acca.at[pl.ds(s * _RPS, _RPS)])
    pltpu.sync_copy(gidx.at[s], gv)
    pltpu.sync_copy(sidx.at[s], sv)
    if not gather_aux:
      pltpu.sync_copy(onesbuf, xbufs[0])
    plsc.subcore_barrier()

    def run_loop(tbl, aux_off):
      for b in range(_NB):
        pltpu.async_copy(tbl.at[gv.at[b]], bufs[b], sem_g[b])
      if gather_aux:
        for a in range(2):
          pltpu.async_copy(auxtbl.at[gv.at[aux_off + 2 * a]],
                           xbufs[a], sem_xg[a])

      def body(t, carry):
        j = 4 * t
        for b in range(_NB):
          pltpu.make_async_copy(tbl.at[gv.at[j + b]],
                                bufs[b], sem_g[b]).wait()
          pltpu.async_copy(bufs[b], accm.at[sv.at[j + b]],
                           sem_s[b], add=True)
        for a in range(2):
          ja = j + aux_off + 2 * a
          src = xbufs[a] if gather_aux else xbufs[0]
          if gather_aux:
            pltpu.make_async_copy(auxtbl.at[gv.at[ja]],
                                  xbufs[a], sem_xg[a]).wait()
          pltpu.async_copy(src, acca.at[sv.at[ja]], sem_xs[a], add=True)
        for b in range(_NB):
          pltpu.make_async_copy(bufs[b], accm.at[sv.at[j + b]],
                                sem_s[b]).wait()
          pltpu.async_copy(tbl.at[gv.at[j + 4 + b]], bufs[b], sem_g[b])
        for a in range(2):
          ja = j + aux_off + 2 * a
          src = xbufs[a] if gather_aux else xbufs[0]
          pltpu.make_async_copy(src, acca.at[sv.at[ja]], sem_xs[a]).wait()
          if gather_aux:
            pltpu.async_copy(auxtbl.at[gv.at[ja + 4]], xbufs[a], sem_xg[a])
        return carry

      lax.fori_loop(0, _CH // 4, body, 0)
      for b in range(_NB):
        pltpu.make_async_copy(tbl.at[gv.at[0]], bufs[b], sem_g[b]).wait()
      if gather_aux:
        for a in range(2):
          pltpu.make_async_copy(auxtbl.at[gv.at[0]],
                                xbufs[a], sem_xg[a]).wait()

    @pl.when(c == 0)
    def _core0():
      run_loop(tbl0, 0)

    @pl.when(c == 1)
    def _core1():
      run_loop(tbl1, 1)

    plsc.subcore_barrier()
    sl = pl.ds(s * _RPS, _RPS)
    pltpu.sync_copy(accm.at[sl], out_main.at[c, sl])
    pltpu.sync_copy(acca.at[sl], out_aux.at[c, sl])

  return run


_pass_edge = _sc_segment_pass(gather_aux=False)
_pass_node = _sc_segment_pass(gather_aux=True)

_BLK = 1000
_GRID = _N // _BLK


@functools.partial(
    pl.pallas_call,
    grid=(_GRID,),
    in_specs=[
        pl.BlockSpec((_BLK, _D), lambda i: (i, 0)),
        pl.BlockSpec((_D, _D), lambda i: (0, 0)),
    ],
    out_specs=[
        pl.BlockSpec((_BLK, _DH), lambda i: (i, 0)),
        pl.BlockSpec((_BLK, _DH), lambda i: (i, 0)),
    ],
    out_shape=[
        jax.ShapeDtypeStruct((_N, _DH), jnp.float32),
        jax.ShapeDtypeStruct((_N, _DH), jnp.float32),
    ],
)
def _matmul(x_ref, w_ref, o0_ref, o1_ref):
  xl = jnp.dot(x_ref[...], w_ref[...], preferred_element_type=jnp.float32)
  o0_ref[...] = xl[:, :_DH]
  o1_ref[...] = xl[:, _DH:]


@functools.partial(
    pl.pallas_call,
    grid=(_GRID,),
    in_specs=[
        pl.BlockSpec((_NC, _BLK, _DH), lambda i: (0, i, 0)),
        pl.BlockSpec((_NC, _BLK, _AUX), lambda i: (0, i, 0)),
        pl.BlockSpec((_BLK, 1), lambda i: (i, 0)),
    ],
    out_specs=[
        pl.BlockSpec((_BLK, _DH), lambda i: (i, 0)),
        pl.BlockSpec((_BLK, _DH), lambda i: (i, 0)),
        pl.BlockSpec((_BLK, _AUX), lambda i: (i, 0)),
    ],
    out_shape=[
        jax.ShapeDtypeStruct((_N, _DH), jnp.float32),
        jax.ShapeDtypeStruct((_N, _DH), jnp.float32),
        jax.ShapeDtypeStruct((_N, _AUX), jnp.float32),
    ],
)
def _combine_mid(pe_ref, pa_ref, w_ref, tn0_ref, tn1_ref, ta_ref):
  bdeg = pa_ref[0, :, 0:1] + pa_ref[1, :, 0:1]
  binv = jnp.where(bdeg > 0, 1.0 / bdeg, 0.0)
  tn0_ref[...] = binv * pe_ref[0]
  tn1_ref[...] = binv * pe_ref[1]
  ta_ref[...] = jnp.broadcast_to(w_ref[...], (_BLK, _AUX))


@functools.partial(
    pl.pallas_call,
    grid=(_GRID,),
    in_specs=[
        pl.BlockSpec((_NC, _BLK, _DH), lambda i: (0, i, 0)),
        pl.BlockSpec((_NC, _BLK, _AUX), lambda i: (0, i, 0)),
        pl.BlockSpec((1, _D), lambda i: (0, 0)),
    ],
    out_specs=pl.BlockSpec((_BLK, _D), lambda i: (i, 0)),
    out_shape=jax.ShapeDtypeStruct((_N, _D), jnp.float32),
)
def _combine_out(pn_ref, pa_ref, b_ref, o_ref):
  deg = pa_ref[0, :, 0:1] + pa_ref[1, :, 0:1]
  dinv = jnp.where(deg > 0, 1.0 / deg, 0.0)
  full = jnp.concatenate([dinv * pn_ref[0], dinv * pn_ref[1]], axis=1)
  o_ref[...] = jnp.maximum(full + b_ref[...], 0.0)


def _pad_idx(idx, fill):
  per_s = idx.reshape(_NS, _PS)
  padded = jnp.pad(per_s, ((0, 0), (0, _PSP - _PS)), constant_values=fill)
  return padded.reshape(_NS, _CHA, _K)


def kernel(x, hyperedge_index, hyperedge_weight, W, b):
  x = x.astype(jnp.float32)
  node_idx = hyperedge_index[0].astype(jnp.int32)
  edge_idx = hyperedge_index[1].astype(jnp.int32)
  node_g = _pad_idx(node_idx, 0)
  node_s = _pad_idx(node_idx, _N)
  edge_g = _pad_idx(edge_idx, 0)
  edge_s = _pad_idx(edge_idx, _N)

  xl0, xl1 = _matmul(x, W.astype(jnp.float32))

  zmain = jnp.zeros((_RPS, _DH), jnp.float32)
  zaux = jnp.zeros((_RPS, _AUX), jnp.float32)
  ones = jnp.ones((_K, _AUX), jnp.float32)
  dummy_aux = jnp.zeros((8, _AUX), jnp.float32)

  pe_main, pe_aux = _pass_edge(xl0, xl1, dummy_aux, node_g, edge_s,
                               zmain, zaux, ones)
  w2 = hyperedge_weight.astype(jnp.float32).reshape(_N, 1)
  tn0, tn1, ta = _combine_mid(pe_main, pe_aux, w2)
  pn_main, pn_aux = _pass_node(tn0, tn1, ta, edge_g, node_s,
                               zmain, zaux, ones)
  return _combine_out(pn_main, pn_aux, b.astype(jnp.float32).reshape(1, _D))


# speedup vs baseline: 1.0304x; 1.0304x over previous
import functools

import jax
import jax.numpy as jnp
from jax import lax
from jax.experimental import pallas as pl
from jax.experimental.pallas import tpu as pltpu
from jax.experimental.pallas import tpu_sc as plsc

_N = 10000
_NNZ = 320000
_D = 128
_DH = _D // 2
_AUX = 16
_NC = 2
_NS = 16
_K = 128
_PS = _NNZ // _NS
_CH = 160
_CHA = _CH + 4
_PSP = _CHA * _K
_NB = 4
_RPS = 632
_NP = _RPS * _NS


def _sc_segment_pass(gather_aux: bool):
  mesh = plsc.VectorSubcoreMesh(core_axis_name="c", subcore_axis_name="s")
  out_type = (
      jax.ShapeDtypeStruct((_NC, _NP, _DH), jnp.float32),
      jax.ShapeDtypeStruct((_NC, _NP, _AUX), jnp.float32),
  )
  scratch = [
      pltpu.VMEM((_CHA, _K), jnp.int32),
      pltpu.VMEM((_CHA, _K), jnp.int32),
      [pltpu.VMEM((_K, _DH), jnp.float32)] * _NB,
      [pltpu.VMEM((_K, _AUX), jnp.float32)] * 2,
      pltpu.VMEM_SHARED((_NP, _DH), jnp.float32),
      pltpu.VMEM_SHARED((_NP, _AUX), jnp.float32),
      [pltpu.SemaphoreType.DMA] * _NB,
      [pltpu.SemaphoreType.DMA] * 2,
  ]

  @functools.partial(
      pl.kernel, out_type=out_type, mesh=mesh, scratch_types=scratch,
      compiler_params=pltpu.CompilerParams(use_tc_tiling_on_sc=False))
  def run(tbl0, tbl1, auxtbl, gidx, sidx, zmain, zaux, onesbuf,
          out_main, out_aux,
          gv, sv, bufs, xbufs, accm, acca, sem_g, sem_xg):
    c = lax.axis_index("c")
    s = lax.axis_index("s")
    pltpu.sync_copy(zmain, accm.at[pl.ds(s * _RPS, _RPS)])
    pltpu.sync_copy(zaux, acca.at[pl.ds(s * _RPS, _RPS)])
    pltpu.sync_copy(gidx.at[s], gv)
    pltpu.sync_copy(sidx.at[s], sv)
    if not gather_aux:
      pltpu.sync_copy(onesbuf, xbufs[0])
    plsc.subcore_barrier()

    def run_loop(tbl, aux_off):
      for b in range(_NB):
        pltpu.async_copy(tbl.at[gv.at[b]], bufs[b], sem_g[b])
      if gather_aux:
        for a in range(2):
          pltpu.async_copy(auxtbl.at[gv.at[aux_off + 2 * a]],
                           xbufs[a], sem_xg[a])

      def body(t, carry):
        j = 4 * t
        for b in range(_NB):
          pltpu.make_async_copy(tbl.at[gv.at[j + b]],
                                bufs[b], sem_g[b]).wait()
          pltpu.sync_copy(bufs[b], accm.at[sv.at[j + b]], add=True)
          pltpu.async_copy(tbl.at[gv.at[j + 4 + b]], bufs[b], sem_g[b])
          if b % 2 == aux_off % 2:
            ja = j + aux_off + (b // 2) * 2
            a = b // 2
            src = xbufs[a] if gather_aux else xbufs[0]
            if gather_aux:
              pltpu.make_async_copy(auxtbl.at[gv.at[ja]],
                                    xbufs[a], sem_xg[a]).wait()
            pltpu.sync_copy(src, acca.at[sv.at[ja]], add=True)
            if gather_aux:
              pltpu.async_copy(auxtbl.at[gv.at[ja + 4]], xbufs[a], sem_xg[a])
        return carry

      lax.fori_loop(0, _CH // 4, body, 0)
      for b in range(_NB):
        pltpu.make_async_copy(tbl.at[gv.at[0]], bufs[b], sem_g[b]).wait()
      if gather_aux:
        for a in range(2):
          pltpu.make_async_copy(auxtbl.at[gv.at[0]],
                                xbufs[a], sem_xg[a]).wait()

    @pl.when(c == 0)
    def _core0():
      run_loop(tbl0, 0)

    @pl.when(c == 1)
    def _core1():
      run_loop(tbl1, 1)

    plsc.subcore_barrier()
    sl = pl.ds(s * _RPS, _RPS)
    pltpu.sync_copy(accm.at[sl], out_main.at[c, sl])
    pltpu.sync_copy(acca.at[sl], out_aux.at[c, sl])

  return run


_pass_edge = _sc_segment_pass(gather_aux=False)
_pass_node = _sc_segment_pass(gather_aux=True)

_BLK = 1000
_GRID = _N // _BLK


@functools.partial(
    pl.pallas_call,
    grid=(_GRID,),
    in_specs=[
        pl.BlockSpec((_BLK, _D), lambda i: (i, 0)),
        pl.BlockSpec((_D, _D), lambda i: (0, 0)),
    ],
    out_specs=[
        pl.BlockSpec((_BLK, _DH), lambda i: (i, 0)),
        pl.BlockSpec((_BLK, _DH), lambda i: (i, 0)),
    ],
    out_shape=[
        jax.ShapeDtypeStruct((_N, _DH), jnp.float32),
        jax.ShapeDtypeStruct((_N, _DH), jnp.float32),
    ],
)
def _matmul(x_ref, w_ref, o0_ref, o1_ref):
  xl = jnp.dot(x_ref[...], w_ref[...], preferred_element_type=jnp.float32)
  o0_ref[...] = xl[:, :_DH]
  o1_ref[...] = xl[:, _DH:]


@functools.partial(
    pl.pallas_call,
    grid=(_GRID,),
    in_specs=[
        pl.BlockSpec((_NC, _BLK, _DH), lambda i: (0, i, 0)),
        pl.BlockSpec((_NC, _BLK, _AUX), lambda i: (0, i, 0)),
        pl.BlockSpec((_BLK, 1), lambda i: (i, 0)),
    ],
    out_specs=[
        pl.BlockSpec((_BLK, _DH), lambda i: (i, 0)),
        pl.BlockSpec((_BLK, _DH), lambda i: (i, 0)),
        pl.BlockSpec((_BLK, _AUX), lambda i: (i, 0)),
    ],
    out_shape=[
        jax.ShapeDtypeStruct((_N, _DH), jnp.float32),
        jax.ShapeDtypeStruct((_N, _DH), jnp.float32),
        jax.ShapeDtypeStruct((_N, _AUX), jnp.float32),
    ],
)
def _combine_mid(pe_ref, pa_ref, w_ref, tn0_ref, tn1_ref, ta_ref):
  bdeg = pa_ref[0, :, 0:1] + pa_ref[1, :, 0:1]
  binv = jnp.where(bdeg > 0, 1.0 / bdeg, 0.0)
  tn0_ref[...] = binv * pe_ref[0]
  tn1_ref[...] = binv * pe_ref[1]
  ta_ref[...] = jnp.broadcast_to(w_ref[...], (_BLK, _AUX))


@functools.partial(
    pl.pallas_call,
    grid=(_GRID,),
    in_specs=[
        pl.BlockSpec((_NC, _BLK, _DH), lambda i: (0, i, 0)),
        pl.BlockSpec((_NC, _BLK, _AUX), lambda i: (0, i, 0)),
        pl.BlockSpec((1, _D), lambda i: (0, 0)),
    ],
    out_specs=pl.BlockSpec((_BLK, _D), lambda i: (i, 0)),
    out_shape=jax.ShapeDtypeStruct((_N, _D), jnp.float32),
)
def _combine_out(pn_ref, pa_ref, b_ref, o_ref):
  deg = pa_ref[0, :, 0:1] + pa_ref[1, :, 0:1]
  dinv = jnp.where(deg > 0, 1.0 / deg, 0.0)
  full = jnp.concatenate([dinv * pn_ref[0], dinv * pn_ref[1]], axis=1)
  o_ref[...] = jnp.maximum(full + b_ref[...], 0.0)


def _pad_idx(idx, fill):
  per_s = idx.reshape(_NS, _PS)
  padded = jnp.pad(per_s, ((0, 0), (0, _PSP - _PS)), constant_values=fill)
  return padded.reshape(_NS, _CHA, _K)


def kernel(x, hyperedge_index, hyperedge_weight, W, b):
  x = x.astype(jnp.float32)
  node_idx = hyperedge_index[0].astype(jnp.int32)
  edge_idx = hyperedge_index[1].astype(jnp.int32)
  node_g = _pad_idx(node_idx, 0)
  node_s = _pad_idx(node_idx, _N)
  edge_g = _pad_idx(edge_idx, 0)
  edge_s = _pad_idx(edge_idx, _N)

  xl0, xl1 = _matmul(x, W.astype(jnp.float32))

  zmain = jnp.zeros((_RPS, _DH), jnp.float32)
  zaux = jnp.zeros((_RPS, _AUX), jnp.float32)
  ones = jnp.ones((_K, _AUX), jnp.float32)
  dummy_aux = jnp.zeros((8, _AUX), jnp.float32)

  pe_main, pe_aux = _pass_edge(xl0, xl1, dummy_aux, node_g, edge_s,
                               zmain, zaux, ones)
  w2 = hyperedge_weight.astype(jnp.float32).reshape(_N, 1)
  tn0, tn1, ta = _combine_mid(pe_main, pe_aux, w2)
  pn_main, pn_aux = _pass_node(tn0, tn1, ta, edge_g, node_s,
                               zmain, zaux, ones)
  return _combine_out(pn_main, pn_aux, b.astype(jnp.float32).reshape(1, _D))


# speedup vs baseline: 1.7351x; 1.6840x over previous
import functools

import jax
import jax.numpy as jnp
from jax import lax
from jax.experimental import pallas as pl
from jax.experimental.pallas import tpu as pltpu
from jax.experimental.pallas import tpu_sc as plsc

_N = 10000
_NNZ = 320000
_D = 128
_DH = _D // 2
_NC = 2
_NS = 16
_L = 16
_K = 128
_QV = _K // _L
_PS = _NNZ // _NS
_CH = 158
_CHA = _CH + 2
_PSP = _CHA * _K
_RPS = 632
_NP = _RPS * _NS


def _sc_edge_pass():
  mesh = plsc.VectorSubcoreMesh(core_axis_name="c", subcore_axis_name="s")
  out_type = (
      jax.ShapeDtypeStruct((_NC, _NP, _DH), jnp.float32),
      jax.ShapeDtypeStruct((_NS, _NP), jnp.float32),
      jax.ShapeDtypeStruct((_NS, _NP), jnp.float32),
  )
  scratch = [
      pltpu.VMEM((_CHA, _K), jnp.int32),
      pltpu.VMEM((_CHA, _K), jnp.int32),
      pltpu.VMEM((_K, _DH), jnp.float32),
      pltpu.VMEM((_K, _DH), jnp.float32),
      pltpu.VMEM((_NP,), jnp.float32),
      pltpu.VMEM((_NP,), jnp.float32),
      pltpu.VMEM_SHARED((_NP, _DH), jnp.float32),
      pltpu.SemaphoreType.DMA,
      pltpu.SemaphoreType.DMA,
  ]

  @functools.partial(
      pl.kernel, out_type=out_type, mesh=mesh, scratch_types=scratch,
      compiler_params=pltpu.CompilerParams(use_tc_tiling_on_sc=False,
                                           needs_layout_passes=False))
  def run(tbl0, tbl1, wvec, zmain, zhist, gidx, sidx,
          out_main, out_b, out_d,
          gv, sv, bufa, bufb, wv, hist, accm, sem_a, sem_b):
    c = lax.axis_index("c")
    s = lax.axis_index("s")
    pltpu.sync_copy(zmain, accm.at[pl.ds(s * _RPS, _RPS)])
    pltpu.sync_copy(gidx.at[s], gv)
    pltpu.sync_copy(sidx.at[s], sv)
    pltpu.sync_copy(wvec, wv)
    pltpu.sync_copy(zhist, hist)
    plsc.subcore_barrier()

    def hist_step(j, weighted):
      svj = sv.at[j]
      gvj = gv.at[j]
      ones = jnp.ones((_L,), jnp.float32)
      for q in range(_QV):
        sl = pl.ds(q * _L, _L)
        eid = svj[sl]
        if weighted:
          wvals = plsc.load_gather(wv, [eid])
          plsc.addupdate_scatter(hist, [gvj[sl]], wvals)
        else:
          plsc.addupdate_scatter(hist, [eid], ones)

    def run_loop(tbl, weighted):
      pltpu.async_copy(tbl.at[gv.at[0]], bufa, sem_a)

      def body(t, carry):
        j0 = 2 * t
        j1 = j0 + 1
        pltpu.async_copy(tbl.at[gv.at[j1]], bufb, sem_b)
        hist_step(j0, weighted)
        pltpu.make_async_copy(tbl.at[gv.at[j0]], bufa, sem_a).wait()
        pltpu.sync_copy(bufa, accm.at[sv.at[j0]], add=True)
        pltpu.async_copy(tbl.at[gv.at[j0 + 2]], bufa, sem_a)
        hist_step(j1, weighted)
        pltpu.make_async_copy(tbl.at[gv.at[j1]], bufb, sem_b).wait()
        pltpu.sync_copy(bufb, accm.at[sv.at[j1]], add=True)
        return carry

      lax.fori_loop(0, _CH // 2, body, 0)
      pltpu.make_async_copy(tbl.at[gv.at[0]], bufa, sem_a).wait()

    @pl.when(c == 0)
    def _core0():
      run_loop(tbl0, weighted=False)

    @pl.when(c == 1)
    def _core1():
      run_loop(tbl1, weighted=True)

    plsc.subcore_barrier()
    sl = pl.ds(s * _RPS, _RPS)
    pltpu.sync_copy(accm.at[sl], out_main.at[c, sl])

    @pl.when(c == 0)
    def _wb_b():
      pltpu.sync_copy(hist, out_b.at[s])

    @pl.when(c == 1)
    def _wb_d():
      pltpu.sync_copy(hist, out_d.at[s])

  return run


def _sc_node_pass():
  mesh = plsc.VectorSubcoreMesh(core_axis_name="c", subcore_axis_name="s")
  out_type = jax.ShapeDtypeStruct((_NC, _NP, _DH), jnp.float32)
  scratch = [
      pltpu.VMEM((_CHA, _K), jnp.int32),
      pltpu.VMEM((_CHA, _K), jnp.int32),
      pltpu.VMEM((_K, _DH), jnp.float32),
      pltpu.VMEM((_K, _DH), jnp.float32),
      pltpu.VMEM_SHARED((_NP, _DH), jnp.float32),
      pltpu.SemaphoreType.DMA,
      pltpu.SemaphoreType.DMA,
  ]

  @functools.partial(
      pl.kernel, out_type=out_type, mesh=mesh, scratch_types=scratch,
      compiler_params=pltpu.CompilerParams(use_tc_tiling_on_sc=False))
  def run(tbl0, tbl1, zmain, gidx, sidx, out_main,
          gv, sv, bufa, bufb, accm, sem_a, sem_b):
    c = lax.axis_index("c")
    s = lax.axis_index("s")
    pltpu.sync_copy(zmain, accm.at[pl.ds(s * _RPS, _RPS)])
    pltpu.sync_copy(gidx.at[s], gv)
    pltpu.sync_copy(sidx.at[s], sv)
    plsc.subcore_barrier()

    def run_loop(tbl):
      pltpu.async_copy(tbl.at[gv.at[0]], bufa, sem_a)

      def body(t, carry):
        j0 = 2 * t
        j1 = j0 + 1
        pltpu.async_copy(tbl.at[gv.at[j1]], bufb, sem_b)
        pltpu.make_async_copy(tbl.at[gv.at[j0]], bufa, sem_a).wait()
        pltpu.sync_copy(bufa, accm.at[sv.at[j0]], add=True)
        pltpu.async_copy(tbl.at[gv.at[j0 + 2]], bufa, sem_a)
        pltpu.make_async_copy(tbl.at[gv.at[j1]], bufb, sem_b).wait()
        pltpu.sync_copy(bufb, accm.at[sv.at[j1]], add=True)
        return carry

      lax.fori_loop(0, _CH // 2, body, 0)
      pltpu.make_async_copy(tbl.at[gv.at[0]], bufa, sem_a).wait()

    @pl.when(c == 0)
    def _core0():
      run_loop(tbl0)

    @pl.when(c == 1)
    def _core1():
      run_loop(tbl1)

    plsc.subcore_barrier()
    sl = pl.ds(s * _RPS, _RPS)
    pltpu.sync_copy(accm.at[sl], out_main.at[c, sl])

  return run


_pass_edge = _sc_edge_pass()
_pass_node = _sc_node_pass()

_BLK = 1000
_GRID = _N // _BLK


@functools.partial(
    pl.pallas_call,
    grid=(_GRID,),
    in_specs=[
        pl.BlockSpec((_BLK, _D), lambda i: (i, 0)),
        pl.BlockSpec((_D, _D), lambda i: (0, 0)),
    ],
    out_specs=[
        pl.BlockSpec((_BLK, _DH), lambda i: (i, 0)),
        pl.BlockSpec((_BLK, _DH), lambda i: (i, 0)),
    ],
    out_shape=[
        jax.ShapeDtypeStruct((_N, _DH), jnp.float32),
        jax.ShapeDtypeStruct((_N, _DH), jnp.float32),
    ],
)
def _matmul(x_ref, w_ref, o0_ref, o1_ref):
  xl = jnp.dot(x_ref[...], w_ref[...], preferred_element_type=jnp.float32)
  o0_ref[...] = xl[:, :_DH]
  o1_ref[...] = xl[:, _DH:]


@functools.partial(
    pl.pallas_call,
    out_shape=[
        jax.ShapeDtypeStruct((_N, _DH), jnp.float32),
        jax.ShapeDtypeStruct((_N, _DH), jnp.float32),
    ],
)
def _combine_mid(pe_ref, pb_ref, tn0_ref, tn1_ref):
  bdeg = jnp.sum(pb_ref[...], axis=0)[:_N, None]
  binv = jnp.where(bdeg > 0, 1.0 / bdeg, 0.0)
  tn0_ref[...] = binv * pe_ref[0, :_N, :]
  tn1_ref[...] = binv * pe_ref[1, :_N, :]


@functools.partial(
    pl.pallas_call,
    out_shape=jax.ShapeDtypeStruct((_N, _D), jnp.float32),
)
def _combine_out(pn_ref, pd_ref, b_ref, o_ref):
  deg = jnp.sum(pd_ref[...], axis=0)[:_N, None]
  dinv = jnp.where(deg > 0, 1.0 / deg, 0.0)
  full = jnp.concatenate(
      [dinv * pn_ref[0, :_N, :], dinv * pn_ref[1, :_N, :]], axis=1)
  o_ref[...] = jnp.maximum(full + b_ref[...], 0.0)


def _pad_idx(idx, fill):
  per_s = idx.reshape(_NS, _PS)
  padded = jnp.pad(per_s, ((0, 0), (0, _PSP - _PS)), constant_values=fill)
  return padded.reshape(_NS, _CHA, _K)


def kernel(x, hyperedge_index, hyperedge_weight, W, b):
  x = x.astype(jnp.float32)
  node_idx = hyperedge_index[0].astype(jnp.int32)
  edge_idx = hyperedge_index[1].astype(jnp.int32)
  node_g = _pad_idx(node_idx, 0)
  node_s = _pad_idx(node_idx, _N)
  edge_g = _pad_idx(edge_idx, 0)
  edge_s = _pad_idx(edge_idx, _N)

  xl0, xl1 = _matmul(x, W.astype(jnp.float32))

  zmain = jnp.zeros((_RPS, _DH), jnp.float32)
  zhist = jnp.zeros((_NP,), jnp.float32)
  wvec = jnp.pad(hyperedge_weight.astype(jnp.float32), (0, _NP - _N))

  pe_main, pb, pd = _pass_edge(xl0, xl1, wvec, zmain, zhist, node_g, edge_s)
  tn0, tn1 = _combine_mid(pe_main, pb)
  pn_main = _pass_node(tn0, tn1, zmain, edge_g, node_s)
  return _combine_out(pn_main, pd, b.astype(jnp.float32).reshape(1, _D))


# speedup vs baseline: 1.7764x; 1.0238x over previous
import functools

import jax
import jax.numpy as jnp
from jax import lax
from jax.experimental import pallas as pl
from jax.experimental.pallas import tpu as pltpu
from jax.experimental.pallas import tpu_sc as plsc

_N = 10000
_NNZ = 320000
_D = 128
_DH = _D // 2
_NC = 2
_NS = 16
_L = 16
_K = 128
_QV = _K // _L
_PS = _NNZ // _NS
_CH = 158
_CHA = _CH + 2
_PSP = _CHA * _K
_RPS = 632
_NP = _RPS * _NS

_SC_PARAMS = pltpu.CompilerParams(use_tc_tiling_on_sc=False,
                                  needs_layout_passes=False)


def _sc_fused_pass():
  mesh = plsc.VectorSubcoreMesh(core_axis_name="c", subcore_axis_name="s")
  out_type = (
      jax.ShapeDtypeStruct((_NC, _NP, _DH), jnp.float32),
      jax.ShapeDtypeStruct((_NS, _NP), jnp.float32),
      jax.ShapeDtypeStruct((_NC, _NP, _DH), jnp.float32),
      jax.ShapeDtypeStruct((_NC, _NS, _NP), jnp.float32),
  )
  scratch = [
      pltpu.VMEM((_CHA, _K), jnp.int32),
      pltpu.VMEM((_CHA, _K), jnp.int32),
      pltpu.VMEM((_K, _DH), jnp.float32),
      pltpu.VMEM((_K, _DH), jnp.float32),
      pltpu.VMEM((_NP,), jnp.float32),
      pltpu.VMEM((_NP,), jnp.float32),
      pltpu.VMEM((_NP,), jnp.float32),
      pltpu.VMEM((_RPS + _L,), jnp.float32),
      pltpu.VMEM((_RPS + _L,), jnp.float32),
      pltpu.VMEM_SHARED((_NP, _DH), jnp.float32),
      pltpu.SemaphoreType.DMA,
      pltpu.SemaphoreType.DMA,
  ]

  @functools.partial(pl.kernel, out_type=out_type, mesh=mesh,
                     scratch_types=scratch, compiler_params=_SC_PARAMS)
  def run(tbl0, tbl1, wvec, zmain, zhist, ng, es, eg, ns,
          out_main, out_d, out_tn, out_bh,
          gv, sv, bufa, bufb, wv, bh, dh, bdeg, tmp,
          acc, sem_a, sem_b):
    c = lax.axis_index("c")
    s = lax.axis_index("s")
    srow = pl.ds(s * _RPS, _RPS)
    pltpu.sync_copy(zmain, acc.at[srow])
    pltpu.sync_copy(ng.at[s], gv)
    pltpu.sync_copy(es.at[s], sv)
    pltpu.sync_copy(wvec, wv)
    pltpu.sync_copy(zhist, bh)
    pltpu.sync_copy(zhist, dh)
    plsc.subcore_barrier()

    def hist_step(j):
      svj = sv.at[j]
      gvj = gv.at[j]
      ones = jnp.ones((_L,), jnp.float32)
      for q in range(_QV):
        sl = pl.ds(q * _L, _L)
        eid = svj[sl]
        plsc.addupdate_scatter(bh, [eid], ones)
        wvals = plsc.load_gather(wv, [eid])
        plsc.addupdate_scatter(dh, [gvj[sl]], wvals)

    def stream_loop(gather_from, dest, with_hist):
      pltpu.async_copy(gather_from.at[gv.at[0]], bufa, sem_a)

      def body(t, carry):
        j0 = 2 * t
        j1 = j0 + 1
        pltpu.async_copy(gather_from.at[gv.at[j1]], bufb, sem_b)
        if with_hist:
          hist_step(j0)
        pltpu.make_async_copy(gather_from.at[gv.at[j0]],
                              bufa, sem_a).wait()
        pltpu.sync_copy(bufa, dest.at[sv.at[j0]], add=True)
        pltpu.async_copy(gather_from.at[gv.at[j0 + 2]], bufa, sem_a)
        if with_hist:
          hist_step(j1)
        pltpu.make_async_copy(gather_from.at[gv.at[j1]],
                              bufb, sem_b).wait()
        pltpu.sync_copy(bufb, dest.at[sv.at[j1]], add=True)
        return carry

      lax.fori_loop(0, _CH // 2, body, 0)
      pltpu.make_async_copy(gather_from.at[gv.at[0]], bufa, sem_a).wait()

    @pl.when(c == 0)
    def _l1c0():
      stream_loop(tbl0, acc, True)

    @pl.when(c == 1)
    def _l1c1():
      stream_loop(tbl1, acc, True)

    plsc.subcore_barrier()
    pltpu.sync_copy(bh, out_bh.at[c, s])
    pltpu.sync_copy(eg.at[s], gv)
    pltpu.sync_copy(ns.at[s], sv)
    plsc.subcore_barrier()

    _NG = (_RPS + _L - 1) // _L
    pltpu.sync_copy(out_bh.at[c, 0, srow], bdeg.at[pl.ds(0, _RPS)])
    for sp in range(1, _NS):
      pltpu.sync_copy(out_bh.at[c, sp, srow], tmp.at[pl.ds(0, _RPS)])
      for q in range(_NG):
        sl = pl.ds(q * _L, _L)
        bdeg[sl] = bdeg[sl] + tmp[sl]
    for q in range(_NG):
      sl = pl.ds(q * _L, _L)
      bv = bdeg[sl]
      bdeg[sl] = jnp.where(bv > 0, 1.0 / bv, 0.0)

    row0 = s * _RPS
    for blk, rows in ((0, 128), (128, 128), (256, 128), (384, 128),
                      (512, 120)):
      seg = pl.ds(row0 + blk, rows)
      pltpu.sync_copy(acc.at[seg], bufa.at[pl.ds(0, rows)])

      def scale_row(r, carry):
        bsc = bdeg[pl.ds(blk + r, _L)][0]
        for q in range(_DH // _L):
          sl = pl.ds(q * _L, _L)
          bufa[r, sl] = bufa[r, sl] * bsc
        return carry

      lax.fori_loop(0, rows, scale_row, 0)
      pltpu.sync_copy(bufa.at[pl.ds(0, rows)], out_tn.at[c, seg])
    pltpu.sync_copy(zmain, acc.at[srow])
    plsc.subcore_barrier()

    @pl.when(c == 0)
    def _l2c0():
      stream_loop(out_tn.at[0], acc, False)

    @pl.when(c == 1)
    def _l2c1():
      stream_loop(out_tn.at[1], acc, False)

    plsc.subcore_barrier()
    pltpu.sync_copy(acc.at[srow], out_main.at[c, srow])

    @pl.when(c == 1)
    def _wb_d():
      pltpu.sync_copy(dh, out_d.at[s])

  return run


_fused = _sc_fused_pass()

_BLK = 1000
_GRID = _N // _BLK


@functools.partial(
    pl.pallas_call,
    grid=(_GRID,),
    in_specs=[
        pl.BlockSpec((_BLK, _D), lambda i: (i, 0)),
        pl.BlockSpec((_D, _D), lambda i: (0, 0)),
    ],
    out_specs=[
        pl.BlockSpec((_BLK, _DH), lambda i: (i, 0)),
        pl.BlockSpec((_BLK, _DH), lambda i: (i, 0)),
    ],
    out_shape=[
        jax.ShapeDtypeStruct((_N, _DH), jnp.float32),
        jax.ShapeDtypeStruct((_N, _DH), jnp.float32),
    ],
)
def _matmul(x_ref, w_ref, o0_ref, o1_ref):
  xl = jnp.dot(x_ref[...], w_ref[...], preferred_element_type=jnp.float32)
  o0_ref[...] = xl[:, :_DH]
  o1_ref[...] = xl[:, _DH:]


@functools.partial(
    pl.pallas_call,
    out_shape=jax.ShapeDtypeStruct((_N, _D), jnp.float32),
)
def _combine_out(pn_ref, pd_ref, b_ref, o_ref):
  deg = jnp.sum(pd_ref[...], axis=0)[:_N, None]
  dinv = jnp.where(deg > 0, 1.0 / deg, 0.0)
  full = jnp.concatenate(
      [dinv * pn_ref[0, :_N, :], dinv * pn_ref[1, :_N, :]], axis=1)
  o_ref[...] = jnp.maximum(full + b_ref[...], 0.0)


def _pad_idx(idx, fill):
  per_s = idx.reshape(_NS, _PS)
  padded = jnp.pad(per_s, ((0, 0), (0, _PSP - _PS)), constant_values=fill)
  return padded.reshape(_NS, _CHA, _K)


def kernel(x, hyperedge_index, hyperedge_weight, W, b):
  x = x.astype(jnp.float32)
  node_idx = hyperedge_index[0].astype(jnp.int32)
  edge_idx = hyperedge_index[1].astype(jnp.int32)
  node_g = _pad_idx(node_idx, 0)
  node_s = _pad_idx(node_idx, _N)
  edge_g = _pad_idx(edge_idx, 0)
  edge_s = _pad_idx(edge_idx, _N)

  xl0, xl1 = _matmul(x, W.astype(jnp.float32))

  zmain = jnp.zeros((_RPS, _DH), jnp.float32)
  zhist = jnp.zeros((_NP,), jnp.float32)
  wvec = jnp.pad(hyperedge_weight.astype(jnp.float32), (0, _NP - _N))

  pn, pd, _, _ = _fused(xl0, xl1, wvec, zmain, zhist,
                        node_g, edge_s, edge_g, node_s)
  return _combine_out(pn, pd, b.astype(jnp.float32).reshape(1, _D))
